# Initial kernel scaffold; baseline (speedup 1.0000x reference)
import jax
import jax.numpy as jnp
from jax.experimental import pallas as pl


def kernel(x, token_table, pos_table):
    raise NotImplementedError("write your pallas kernel here")


# speedup vs baseline: 1.1174x; 1.1174x over previous
import functools
import math

import jax
import jax.numpy as jnp
from jax import lax
from jax.experimental import pallas as pl
from jax.experimental.pallas import tpu as pltpu
from jax.experimental.pallas import tpu_sc as plsc

VOCAB = 100000
EMBED_DIM = 128
BATCH = 4
SEQ_LEN = 4096
TOTAL = BATCH * SEQ_LEN
SCALE = math.sqrt(EMBED_DIM)

_info = plsc.get_sparse_core_info()
NC, NS, L = _info.num_cores, _info.num_subcores, _info.num_lanes
NW = NC * NS
PER_W = TOTAL // NW
CHUNK = 128
NCHUNK = PER_W // CHUNK
LANES = EMBED_DIM // 16


def _body(x_hbm, tok_hbm, pos_hbm, out_hbm, idx_v, tok_v, pos_v, gsem, psem):
    wid = lax.axis_index("s") * NC + lax.axis_index("c")
    base = wid * PER_W
    pos_start = lax.rem(base, SEQ_LEN)

    pltpu.sync_copy(x_hbm.at[pl.ds(wid * NCHUNK, NCHUNK)], idx_v)

    for c in range(NCHUNK):
        gcp = pltpu.async_copy(tok_hbm.at[idx_v.at[c]], tok_v, gsem)
        pcp = pltpu.async_copy(
            pos_hbm.at[pl.ds(pos_start + c * CHUNK, CHUNK)], pos_v, psem)
        gcp.wait()
        pcp.wait()

        def row_body(r, _):
            for cc in range(LANES):
                sl = pl.ds(cc * 16, 16)
                tok_v[r, sl] = tok_v[r, sl] * SCALE + pos_v[r, sl]
            return 0

        lax.fori_loop(0, CHUNK, row_body, 0)
        pltpu.sync_copy(tok_v, out_hbm.at[pl.ds(base + c * CHUNK, CHUNK)])


@jax.jit
def kernel(x, token_table, pos_table):
    x2 = x.reshape(TOTAL // CHUNK, CHUNK).astype(jnp.int32)
    mesh = plsc.VectorSubcoreMesh(core_axis_name="c", subcore_axis_name="s")
    run = pl.kernel(
        _body,
        out_type=jax.ShapeDtypeStruct((TOTAL, EMBED_DIM), jnp.float32),
        mesh=mesh,
        scratch_types=[
            pltpu.VMEM((NCHUNK, CHUNK), jnp.int32),
            pltpu.VMEM((CHUNK, EMBED_DIM), jnp.float32),
            pltpu.VMEM((CHUNK, EMBED_DIM), jnp.float32),
            pltpu.SemaphoreType.DMA,
            pltpu.SemaphoreType.DMA,
        ],
    )
    out = run(x2, token_table, pos_table)
    return out.reshape(BATCH, SEQ_LEN, EMBED_DIM)


# speedup vs baseline: 1.3701x; 1.2261x over previous
import math

import jax
import jax.numpy as jnp
from jax import lax
from jax.experimental import pallas as pl
from jax.experimental.pallas import tpu as pltpu
from jax.experimental.pallas import tpu_sc as plsc

VOCAB = 100000
EMBED_DIM = 128
BATCH = 4
SEQ_LEN = 4096
TOTAL = BATCH * SEQ_LEN
SCALE = math.sqrt(EMBED_DIM)

_info = plsc.get_sparse_core_info()
NC, NS = _info.num_cores, _info.num_subcores
NW = NC * NS
CHUNK = SEQ_LEN // NW
LANES = EMBED_DIM // 16


def _body(x_hbm, tok_hbm, pos_hbm, out_hbm,
          idx_v, pos_v, tok0_v, tok1_v, res0_v, res1_v,
          psem, gsem0, gsem1, osem0, osem1):
    wid = lax.axis_index("s") * NC + lax.axis_index("c")
    tok_bufs = (tok0_v, tok1_v)
    res_bufs = (res0_v, res1_v)
    gsems = (gsem0, gsem1)
    osems = (osem0, osem1)

    pltpu.sync_copy(x_hbm.at[wid], idx_v)
    pcp = pltpu.async_copy(pos_hbm.at[pl.ds(wid * CHUNK, CHUNK)], pos_v, psem)

    gcps = [None] * BATCH
    for c in range(min(2, BATCH)):
        gcps[c] = pltpu.async_copy(
            tok_hbm.at[idx_v.at[c]], tok_bufs[c & 1], gsems[c & 1])

    pcp.wait()
    ocps = [None] * BATCH
    for c in range(BATCH):
        b = c & 1
        gcps[c].wait()
        if ocps[c - 2] is not None:
            ocps[c - 2].wait()

        tok_v = tok_bufs[b]
        res_v = res_bufs[b]

        def row_body(r, _):
            for cc in range(LANES):
                sl = pl.ds(cc * 16, 16)
                res_v[r, sl] = tok_v[r, sl] * SCALE + pos_v[r, sl]
            return 0

        lax.fori_loop(0, CHUNK, row_body, 0)

        if c + 2 < BATCH:
            gcps[c + 2] = pltpu.async_copy(
                tok_hbm.at[idx_v.at[c + 2]], tok_bufs[b], gsems[b])
        ocps[c] = pltpu.async_copy(
            res_v, out_hbm.at[pl.ds(c * SEQ_LEN + wid * CHUNK, CHUNK)],
            osems[b])
    ocps[BATCH - 2].wait()
    ocps[BATCH - 1].wait()


@jax.jit
def kernel(x, token_table, pos_table):
    x_t = x.astype(jnp.int32).reshape(BATCH, NW, CHUNK).transpose(1, 0, 2)
    mesh = plsc.VectorSubcoreMesh(core_axis_name="c", subcore_axis_name="s")
    run = pl.kernel(
        _body,
        out_type=jax.ShapeDtypeStruct((TOTAL, EMBED_DIM), jnp.float32),
        mesh=mesh,
        scratch_types=[
            pltpu.VMEM((BATCH, CHUNK), jnp.int32),
            pltpu.VMEM((CHUNK, EMBED_DIM), jnp.float32),
            pltpu.VMEM((CHUNK, EMBED_DIM), jnp.float32),
            pltpu.VMEM((CHUNK, EMBED_DIM), jnp.float32),
            pltpu.VMEM((CHUNK, EMBED_DIM), jnp.float32),
            pltpu.VMEM((CHUNK, EMBED_DIM), jnp.float32),
            pltpu.SemaphoreType.DMA,
            pltpu.SemaphoreType.DMA,
            pltpu.SemaphoreType.DMA,
            pltpu.SemaphoreType.DMA,
            pltpu.SemaphoreType.DMA,
        ],
    )
    out = run(x_t, token_table, pos_table)
    return out.reshape(BATCH, SEQ_LEN, EMBED_DIM)


# speedup vs baseline: 1.3851x; 1.0110x over previous
import math

import jax
import jax.numpy as jnp
from jax import lax
from jax.experimental import pallas as pl
from jax.experimental.pallas import tpu as pltpu
from jax.experimental.pallas import tpu_sc as plsc

VOCAB = 100000
EMBED_DIM = 128
BATCH = 4
SEQ_LEN = 4096
TOTAL = BATCH * SEQ_LEN
SCALE = math.sqrt(EMBED_DIM)

_info = plsc.get_sparse_core_info()
NC, NS = _info.num_cores, _info.num_subcores
NW = NC * NS
CHUNK = SEQ_LEN // NW
LANES = EMBED_DIM // 16


def _body(x_hbm, tok_hbm, pos_hbm, out_hbm,
          idx_v, pos_v, tok0_v, tok1_v, tok2_v, tok3_v,
          isem, psem, gsem0, gsem1, gsem2, gsem3, osem0, osem1, osem2, osem3):
    wid = lax.axis_index("s") * NC + lax.axis_index("c")
    tok_bufs = (tok0_v, tok1_v, tok2_v, tok3_v)
    gsems = (gsem0, gsem1, gsem2, gsem3)
    osems = (osem0, osem1, osem2, osem3)

    pcp = pltpu.async_copy(pos_hbm.at[pl.ds(wid * CHUNK, CHUNK)], pos_v, psem)

    icps = [pltpu.async_copy(x_hbm.at[b, pl.ds(wid * CHUNK, CHUNK)],
                             idx_v.at[b], isem)
            for b in range(BATCH)]
    for icp in icps:
        icp.wait()

    gcps = [pltpu.async_copy(tok_hbm.at[idx_v.at[c]], tok_bufs[c], gsems[c])
            for c in range(BATCH)]

    pcp.wait()
    ocps = []
    for c in range(BATCH):
        gcps[c].wait()
        tok_v = tok_bufs[c]

        def row_body(r, _):
            for cc in range(LANES):
                sl = pl.ds(cc * 16, 16)
                tok_v[r, sl] = tok_v[r, sl] * SCALE + pos_v[r, sl]
            return 0

        lax.fori_loop(0, CHUNK, row_body, 0)
        ocps.append(pltpu.async_copy(
            tok_v, out_hbm.at[pl.ds(c * SEQ_LEN + wid * CHUNK, CHUNK)],
            osems[c]))
    for ocp in ocps:
        ocp.wait()


@jax.jit
def kernel(x, token_table, pos_table):
    mesh = plsc.VectorSubcoreMesh(core_axis_name="c", subcore_axis_name="s")
    run = pl.kernel(
        _body,
        out_type=jax.ShapeDtypeStruct((TOTAL, EMBED_DIM), jnp.float32),
        mesh=mesh,
        scratch_types=[
            pltpu.VMEM((BATCH, CHUNK), jnp.int32),
            pltpu.VMEM((CHUNK, EMBED_DIM), jnp.float32),
            pltpu.VMEM((CHUNK, EMBED_DIM), jnp.float32),
            pltpu.VMEM((CHUNK, EMBED_DIM), jnp.float32),
            pltpu.VMEM((CHUNK, EMBED_DIM), jnp.float32),
            pltpu.VMEM((CHUNK, EMBED_DIM), jnp.float32),
        ] + [pltpu.SemaphoreType.DMA] * 10,
    )
    out = run(x.astype(jnp.int32), token_table, pos_table)
    return out.reshape(BATCH, SEQ_LEN, EMBED_DIM)
